# Initial kernel scaffold; baseline (speedup 1.0000x reference)
import jax
import jax.numpy as jnp
from jax.experimental import pallas as pl


def kernel(x, edge_index, batch, W1, b1, W2, b2, W3, b3, W4, b4, L1W, L1b, L2W, L2b):
    raise NotImplementedError("write your pallas kernel here")


# speedup vs baseline: 9.6168x; 9.6168x over previous
import functools

import jax
import jax.numpy as jnp
from jax import lax
from jax.experimental import pallas as pl
from jax.experimental.pallas import tpu as pltpu
from jax.experimental.pallas import tpu_sc as plsc

NC = 2
NS = 16

_MESH = plsc.VectorSubcoreMesh(
    core_axis_name="c", subcore_axis_name="s", num_cores=NC, num_subcores=NS)

_DOT = functools.partial(
    jnp.dot, preferred_element_type=jnp.float32,
    precision=jax.lax.Precision.HIGHEST)


def _sc_degree(dst, zeros_n):
  e = dst.shape[0]
  n = zeros_n.shape[0]
  per_w = e // (NC * NS)
  chunk = 80
  nch = per_w // chunk
  assert per_w % chunk == 0 and per_w % 8 == 0

  @functools.partial(
      pl.kernel,
      out_type=[jax.ShapeDtypeStruct((n,), jnp.float32),
                jax.ShapeDtypeStruct((n,), jnp.float32)],
      mesh=_MESH,
      scratch_types=[
          pltpu.VMEM((chunk,), jnp.int32),
          pltpu.VMEM((chunk,), jnp.float32),
          pltpu.VMEM_SHARED((n,), jnp.float32),
      ],
  )
  def deg_kernel(dst_hbm, z_hbm, out0_hbm, out1_hbm, idx_v, ones_v, acc_sh):
    c = lax.axis_index("c")
    s = lax.axis_index("s")
    for j in range(chunk // 16):
      ones_v[pl.ds(j * 16, 16)] = jnp.full((16,), 1.0, jnp.float32)

    @pl.when(s == 0)
    def _():
      pltpu.sync_copy(z_hbm, acc_sh)
    plsc.subcore_barrier()

    wbase = (c * NS + s) * per_w

    def body(i, carry):
      base = pl.multiple_of(wbase + i * chunk, 8)
      pltpu.sync_copy(dst_hbm.at[pl.ds(base, chunk)], idx_v)
      pltpu.sync_copy(ones_v, acc_sh.at[idx_v], add=True)
      return carry

    lax.fori_loop(0, nch, body, 0)
    plsc.subcore_barrier()

    @pl.when((s == 0) & (c == 0))
    def _():
      pltpu.sync_copy(acc_sh, out0_hbm)

    @pl.when((s == 0) & (c == 1))
    def _():
      pltpu.sync_copy(acc_sh, out1_hbm)

  d0, d1 = deg_kernel(dst, zeros_n)
  return jnp.stack([d0, d1])


def _sc_agg(y, src, dst, zeros_nh):
  n, h = y.shape
  e = src.shape[0]
  per_w = e // (NC * NS)
  chunk = 80
  nch = per_w // chunk
  assert per_w % chunk == 0
  rows_pt = n // NS

  @functools.partial(
      pl.kernel,
      out_type=[jax.ShapeDtypeStruct((n, h), jnp.float32),
                jax.ShapeDtypeStruct((n, h), jnp.float32)],
      mesh=_MESH,
      scratch_types=[
          pltpu.VMEM((chunk,), jnp.int32),
          pltpu.VMEM((chunk,), jnp.int32),
          pltpu.VMEM((chunk, h), jnp.float32),
          pltpu.VMEM_SHARED((n, h), jnp.float32),
          pltpu.SemaphoreType.DMA,
      ],
  )
  def agg_kernel(y_hbm, src_hbm, dst_hbm, z_hbm, out0_hbm, out1_hbm,
                 si_v, di_v, rows_v, acc_sh, sem):
    c = lax.axis_index("c")
    s = lax.axis_index("s")
    rw = (rows_pt // 8 + 1) * 8
    r0 = pl.multiple_of(s * rows_pt // 8 * 8, 8)
    pltpu.sync_copy(z_hbm.at[pl.ds(r0, rw)], acc_sh.at[pl.ds(r0, rw)])
    plsc.subcore_barrier()

    wbase = (c * NS + s) * per_w

    def body(i, carry):
      base = pl.multiple_of(wbase + i * chunk, 8)
      pltpu.sync_copy(src_hbm.at[pl.ds(base, chunk)], si_v)
      pltpu.sync_copy(dst_hbm.at[pl.ds(base, chunk)], di_v)
      pltpu.async_copy(y_hbm.at[si_v], rows_v, sem).wait()
      pltpu.sync_copy(rows_v, acc_sh.at[di_v], add=True)
      return carry

    lax.fori_loop(0, nch, body, 0)
    plsc.subcore_barrier()

    @pl.when(c == 0)
    def _():
      pltpu.sync_copy(acc_sh.at[pl.ds(r0, rw)], out0_hbm.at[pl.ds(r0, rw)])

    @pl.when(c == 1)
    def _():
      pltpu.sync_copy(acc_sh.at[pl.ds(r0, rw)], out1_hbm.at[pl.ds(r0, rw)])

  return agg_kernel(y, src, dst, zeros_nh)


def _tc_dis(degp):
  n = degp.shape[1]

  def body(d_ref, o_ref):
    d = d_ref[...]
    o_ref[...] = lax.rsqrt(1.0 + d[0, :] + d[1, :])[:, None]

  return pl.pallas_call(
      body, out_shape=jax.ShapeDtypeStruct((n, 1), jnp.float32))(degp)


def _tc_first(x, dis, w1, bat3, bn):
  n, d = x.shape
  h = w1.shape[1]
  g = 64
  nb = n // bn

  def body(x_ref, dis_ref, w_ref, b3_ref, y_ref, cnt_ref):
    i = pl.program_id(0)
    y_ref[...] = dis_ref[...] * _DOT(x_ref[...], w_ref[...])
    bb = b3_ref[0, 0, :]
    ids = lax.broadcasted_iota(jnp.int32, (g, bn), 0)
    m = (ids == bb[None, :]).astype(jnp.float32)

    @pl.when(i == 0)
    def _():
      cnt_ref[...] = jnp.zeros_like(cnt_ref)
    cnt_ref[...] += jnp.sum(m, axis=1, keepdims=True)

  return pl.pallas_call(
      body,
      grid=(nb,),
      in_specs=[
          pl.BlockSpec((bn, d), lambda i: (i, 0)),
          pl.BlockSpec((bn, 1), lambda i: (i, 0)),
          pl.BlockSpec((d, h), lambda i: (0, 0)),
          pl.BlockSpec((1, 1, bn), lambda i: (i, 0, 0)),
      ],
      out_specs=[
          pl.BlockSpec((bn, h), lambda i: (i, 0)),
          pl.BlockSpec((g, 1), lambda i: (0, 0)),
      ],
      out_shape=[
          jax.ShapeDtypeStruct((n, h), jnp.float32),
          jax.ShapeDtypeStruct((g, 1), jnp.float32),
      ],
  )(x, dis, w1, bat3)


def _tc_mid(a0, a1, y, dis, bias, w_next, bat3, bn):
  n, h = y.shape
  g = 64
  nb = n // bn

  def body(a0_ref, a1_ref, y_ref, dis_ref, b_ref, w_ref, b3_ref,
           y2_ref, ps_ref):
    i = pl.program_id(0)
    dd = dis_ref[...]
    hh = jnp.maximum(
        dd * (a0_ref[...] + a1_ref[...] + y_ref[...]) + b_ref[...], 0.0)
    bb = b3_ref[0, 0, :]
    ids = lax.broadcasted_iota(jnp.int32, (g, bn), 0)
    m = (ids == bb[None, :]).astype(jnp.float32)

    @pl.when(i == 0)
    def _():
      ps_ref[...] = jnp.zeros_like(ps_ref)
    ps_ref[...] += _DOT(m, hh)

    y2_ref[...] = dd * _DOT(hh, w_ref[...])

  return pl.pallas_call(
      body,
      grid=(nb,),
      in_specs=[
          pl.BlockSpec((bn, h), lambda i: (i, 0)),
          pl.BlockSpec((bn, h), lambda i: (i, 0)),
          pl.BlockSpec((bn, h), lambda i: (i, 0)),
          pl.BlockSpec((bn, 1), lambda i: (i, 0)),
          pl.BlockSpec((1, h), lambda i: (0, 0)),
          pl.BlockSpec((h, h), lambda i: (0, 0)),
          pl.BlockSpec((1, 1, bn), lambda i: (i, 0, 0)),
      ],
      out_specs=[
          pl.BlockSpec((bn, h), lambda i: (i, 0)),
          pl.BlockSpec((g, h), lambda i: (0, 0)),
      ],
      out_shape=[
          jax.ShapeDtypeStruct((n, h), jnp.float32),
          jax.ShapeDtypeStruct((g, h), jnp.float32),
      ],
  )(a0, a1, y, dis, bias, w_next, bat3)


def _tc_last(a0, a1, y, dis, bias, bat3, bn):
  n, h = y.shape
  g = 64
  nb = n // bn

  def body(a0_ref, a1_ref, y_ref, dis_ref, b_ref, b3_ref, ps_ref):
    i = pl.program_id(0)
    hh = jnp.maximum(
        dis_ref[...] * (a0_ref[...] + a1_ref[...] + y_ref[...]) + b_ref[...],
        0.0)
    bb = b3_ref[0, 0, :]
    ids = lax.broadcasted_iota(jnp.int32, (g, bn), 0)
    m = (ids == bb[None, :]).astype(jnp.float32)

    @pl.when(i == 0)
    def _():
      ps_ref[...] = jnp.zeros_like(ps_ref)
    ps_ref[...] += _DOT(m, hh)

  return pl.pallas_call(
      body,
      grid=(nb,),
      in_specs=[
          pl.BlockSpec((bn, h), lambda i: (i, 0)),
          pl.BlockSpec((bn, h), lambda i: (i, 0)),
          pl.BlockSpec((bn, h), lambda i: (i, 0)),
          pl.BlockSpec((bn, 1), lambda i: (i, 0)),
          pl.BlockSpec((1, h), lambda i: (0, 0)),
          pl.BlockSpec((1, 1, bn), lambda i: (i, 0, 0)),
      ],
      out_specs=pl.BlockSpec((g, h), lambda i: (0, 0)),
      out_shape=jax.ShapeDtypeStruct((g, h), jnp.float32),
  )(a0, a1, y, dis, bias, bat3)


def _tc_mlp(ps, cnt, l1w, l1b, l2w, l2b):
  g = ps.shape[1]

  def body(ps_ref, cnt_ref, w1_ref, b1_ref, w2_ref, b2_ref, o_ref):
    p = ps_ref[...]
    inv = 1.0 / jnp.maximum(cnt_ref[...], 1.0)
    hcat = jnp.concatenate([p[l] for l in range(p.shape[0])], axis=1) * inv
    t = jnp.maximum(_DOT(hcat, w1_ref[...]) + b1_ref[...], 0.0)
    o_ref[...] = _DOT(t, w2_ref[...]) + b2_ref[...]

  return pl.pallas_call(
      body, out_shape=jax.ShapeDtypeStruct((g, 1), jnp.float32),
  )(ps, cnt, l1w, l1b, l2w, l2b)


def kernel(x, edge_index, batch, W1, b1, W2, b2, W3, b3, W4, b4,
           L1W, L1b, L2W, L2b):
  n, d = x.shape
  h = W1.shape[1]
  bn = 1000
  src = edge_index[0]
  dst = edge_index[1]
  bat3 = batch.reshape(n // bn, 1, bn)
  zeros_n = jnp.zeros((n,), jnp.float32)
  zeros_nh = jnp.zeros((n, h), jnp.float32)

  degp = _sc_degree(dst, zeros_n)
  dis = _tc_dis(degp)

  y, cnt = _tc_first(x, dis, W1, bat3, bn)

  biases = [b1.reshape(1, h), b2.reshape(1, h), b3.reshape(1, h),
            b4.reshape(1, h), b4.reshape(1, h)]
  wnexts = [W2, W3, W4, W4]

  psums = []
  for l in range(4):
    a0, a1 = _sc_agg(y, src, dst, zeros_nh)
    y, ps = _tc_mid(a0, a1, y, dis, biases[l], wnexts[l], bat3, bn)
    psums.append(ps)

  a0, a1 = _sc_agg(y, src, dst, zeros_nh)
  psums.append(_tc_last(a0, a1, y, dis, biases[4], bat3, bn))

  out = _tc_mlp(jnp.stack(psums), cnt, L1W, L1b.reshape(1, -1),
                L2W, L2b.reshape(1, 1))
  return out.reshape(-1)


# speedup vs baseline: 19.7379x; 2.0524x over previous
import functools

import jax
import jax.numpy as jnp
from jax import lax
from jax.experimental import pallas as pl
from jax.experimental.pallas import tpu as pltpu
from jax.experimental.pallas import tpu_sc as plsc

NC = 2
NS = 16

_MESH = plsc.VectorSubcoreMesh(
    core_axis_name="c", subcore_axis_name="s", num_cores=NC, num_subcores=NS)

_DOT = functools.partial(
    jnp.dot, preferred_element_type=jnp.float32,
    precision=jax.lax.Precision.HIGHEST)


def _sc_degree(dst, zeros_n):
  e = dst.shape[0]
  n = zeros_n.shape[0]
  per_w = e // (NC * NS)
  chunk = 80
  nch = per_w // chunk
  assert per_w % chunk == 0 and per_w % 8 == 0

  @functools.partial(
      pl.kernel,
      out_type=[jax.ShapeDtypeStruct((n,), jnp.float32),
                jax.ShapeDtypeStruct((n,), jnp.float32)],
      mesh=_MESH,
      scratch_types=[
          pltpu.VMEM((chunk,), jnp.int32),
          pltpu.VMEM((chunk,), jnp.float32),
          pltpu.VMEM_SHARED((n,), jnp.float32),
      ],
  )
  def deg_kernel(dst_hbm, z_hbm, out0_hbm, out1_hbm, idx_v, ones_v, acc_sh):
    c = lax.axis_index("c")
    s = lax.axis_index("s")
    for j in range(chunk // 16):
      ones_v[pl.ds(j * 16, 16)] = jnp.full((16,), 1.0, jnp.float32)

    @pl.when(s == 0)
    def _():
      pltpu.sync_copy(z_hbm, acc_sh)
    plsc.subcore_barrier()

    wbase = (c * NS + s) * per_w

    def body(i, carry):
      base = pl.multiple_of(wbase + i * chunk, 8)
      pltpu.sync_copy(dst_hbm.at[pl.ds(base, chunk)], idx_v)
      pltpu.sync_copy(ones_v, acc_sh.at[idx_v], add=True)
      return carry

    lax.fori_loop(0, nch, body, 0)
    plsc.subcore_barrier()

    @pl.when((s == 0) & (c == 0))
    def _():
      pltpu.sync_copy(acc_sh, out0_hbm)

    @pl.when((s == 0) & (c == 1))
    def _():
      pltpu.sync_copy(acc_sh, out1_hbm)

  d0, d1 = deg_kernel(dst, zeros_n)
  return jnp.stack([d0, d1])


def _sc_agg(y, src, dst, zeros_nh):
  n, h = y.shape
  nrow, _, chunk = src.shape
  per_w = nrow // (NC * NS)
  assert per_w % 2 == 0
  nh = per_w // 2
  rows_pt = n // NS

  @functools.partial(
      pl.kernel,
      out_type=[jax.ShapeDtypeStruct((n, h), jnp.float32),
                jax.ShapeDtypeStruct((n, h), jnp.float32)],
      mesh=_MESH,
      scratch_types=[
          pltpu.VMEM((4, 1, chunk), jnp.int32),
          pltpu.VMEM((4, 1, chunk), jnp.int32),
          pltpu.VMEM((chunk, h), jnp.float32),
          pltpu.VMEM((chunk, h), jnp.float32),
          pltpu.VMEM_SHARED((n, h), jnp.float32),
          pltpu.SemaphoreType.DMA,
          pltpu.SemaphoreType.DMA,
          pltpu.SemaphoreType.DMA,
      ],
  )
  def agg_kernel(y_hbm, src_hbm, dst_hbm, z_hbm, out0_hbm, out1_hbm,
                 si_v, di_v, rows0_v, rows1_v, acc_sh, sem0, sem1, semi):
    c = lax.axis_index("c")
    s = lax.axis_index("s")
    rw = (rows_pt // 8 + 1) * 8
    r0 = pl.multiple_of(s * rows_pt // 8 * 8, 8)
    pltpu.sync_copy(z_hbm.at[pl.ds(r0, rw)], acc_sh.at[pl.ds(r0, rw)])

    wr = (c * NS + s) * per_w
    pltpu.sync_copy(src_hbm.at[pl.ds(wr, 2)], si_v.at[pl.ds(0, 2)])
    pltpu.sync_copy(dst_hbm.at[pl.ds(wr, 2)], di_v.at[pl.ds(0, 2)])
    plsc.subcore_barrier()

    pltpu.async_copy(y_hbm.at[si_v.at[0, 0]], rows0_v, sem0)

    def body(k, carry):
      j = 2 * k
      s0 = j % 4
      s1 = (j + 1) % 4
      sp = (j + 2) % 4

      pltpu.make_async_copy(y_hbm.at[si_v.at[s0, 0]], rows0_v, sem0).wait()
      pltpu.async_copy(y_hbm.at[si_v.at[s1, 0]], rows1_v, sem1)

      @pl.when(k < nh - 1)
      def _():
        pltpu.async_copy(src_hbm.at[pl.ds(wr + j + 2, 2)],
                         si_v.at[pl.ds(sp, 2)], semi)
        pltpu.async_copy(dst_hbm.at[pl.ds(wr + j + 2, 2)],
                         di_v.at[pl.ds(sp, 2)], semi)

      pltpu.sync_copy(rows0_v, acc_sh.at[di_v.at[s0, 0]], add=True)
      pltpu.make_async_copy(y_hbm.at[si_v.at[s1, 0]], rows1_v, sem1).wait()

      @pl.when(k < nh - 1)
      def _():
        pltpu.make_async_copy(src_hbm.at[pl.ds(wr + j + 2, 2)],
                              si_v.at[pl.ds(sp, 2)], semi).wait()
        pltpu.make_async_copy(dst_hbm.at[pl.ds(wr + j + 2, 2)],
                              di_v.at[pl.ds(sp, 2)], semi).wait()
        pltpu.async_copy(y_hbm.at[si_v.at[sp, 0]], rows0_v, sem0)

      pltpu.sync_copy(rows1_v, acc_sh.at[di_v.at[s1, 0]], add=True)
      return carry

    lax.fori_loop(0, nh, body, 0)
    plsc.subcore_barrier()

    @pl.when(c == 0)
    def _():
      pltpu.sync_copy(acc_sh.at[pl.ds(r0, rw)], out0_hbm.at[pl.ds(r0, rw)])

    @pl.when(c == 1)
    def _():
      pltpu.sync_copy(acc_sh.at[pl.ds(r0, rw)], out1_hbm.at[pl.ds(r0, rw)])

  return agg_kernel(y, src, dst, zeros_nh)


def _tc_dis(degp):
  n = degp.shape[1]

  def body(d_ref, o_ref):
    d = d_ref[...]
    o_ref[...] = lax.rsqrt(1.0 + d[0, :] + d[1, :])[:, None]

  return pl.pallas_call(
      body, out_shape=jax.ShapeDtypeStruct((n, 1), jnp.float32))(degp)


def _tc_first(x, dis, w1, bat3, bn):
  n, d = x.shape
  h = w1.shape[1]
  g = 64
  nb = n // bn

  def body(x_ref, dis_ref, w_ref, b3_ref, y_ref, cnt_ref):
    i = pl.program_id(0)
    y_ref[...] = dis_ref[...] * _DOT(x_ref[...], w_ref[...])
    bb = b3_ref[0, 0, :]
    ids = lax.broadcasted_iota(jnp.int32, (g, bn), 0)
    m = (ids == bb[None, :]).astype(jnp.float32)

    @pl.when(i == 0)
    def _():
      cnt_ref[...] = jnp.zeros_like(cnt_ref)
    cnt_ref[...] += jnp.sum(m, axis=1, keepdims=True)

  return pl.pallas_call(
      body,
      grid=(nb,),
      in_specs=[
          pl.BlockSpec((bn, d), lambda i: (i, 0)),
          pl.BlockSpec((bn, 1), lambda i: (i, 0)),
          pl.BlockSpec((d, h), lambda i: (0, 0)),
          pl.BlockSpec((1, 1, bn), lambda i: (i, 0, 0)),
      ],
      out_specs=[
          pl.BlockSpec((bn, h), lambda i: (i, 0)),
          pl.BlockSpec((g, 1), lambda i: (0, 0)),
      ],
      out_shape=[
          jax.ShapeDtypeStruct((n, h), jnp.float32),
          jax.ShapeDtypeStruct((g, 1), jnp.float32),
      ],
  )(x, dis, w1, bat3)


def _tc_mid(a0, a1, y, dis, bias, w_next, bat3, bn):
  n, h = y.shape
  g = 64
  nb = n // bn

  def body(a0_ref, a1_ref, y_ref, dis_ref, b_ref, w_ref, b3_ref,
           y2_ref, ps_ref):
    i = pl.program_id(0)
    dd = dis_ref[...]
    hh = jnp.maximum(
        dd * (a0_ref[...] + a1_ref[...] + y_ref[...]) + b_ref[...], 0.0)
    bb = b3_ref[0, 0, :]
    ids = lax.broadcasted_iota(jnp.int32, (g, bn), 0)
    m = (ids == bb[None, :]).astype(jnp.float32)

    @pl.when(i == 0)
    def _():
      ps_ref[...] = jnp.zeros_like(ps_ref)
    ps_ref[...] += _DOT(m, hh)

    y2_ref[...] = dd * _DOT(hh, w_ref[...])

  return pl.pallas_call(
      body,
      grid=(nb,),
      in_specs=[
          pl.BlockSpec((bn, h), lambda i: (i, 0)),
          pl.BlockSpec((bn, h), lambda i: (i, 0)),
          pl.BlockSpec((bn, h), lambda i: (i, 0)),
          pl.BlockSpec((bn, 1), lambda i: (i, 0)),
          pl.BlockSpec((1, h), lambda i: (0, 0)),
          pl.BlockSpec((h, h), lambda i: (0, 0)),
          pl.BlockSpec((1, 1, bn), lambda i: (i, 0, 0)),
      ],
      out_specs=[
          pl.BlockSpec((bn, h), lambda i: (i, 0)),
          pl.BlockSpec((g, h), lambda i: (0, 0)),
      ],
      out_shape=[
          jax.ShapeDtypeStruct((n, h), jnp.float32),
          jax.ShapeDtypeStruct((g, h), jnp.float32),
      ],
  )(a0, a1, y, dis, bias, w_next, bat3)


def _tc_last(a0, a1, y, dis, bias, bat3, bn):
  n, h = y.shape
  g = 64
  nb = n // bn

  def body(a0_ref, a1_ref, y_ref, dis_ref, b_ref, b3_ref, ps_ref):
    i = pl.program_id(0)
    hh = jnp.maximum(
        dis_ref[...] * (a0_ref[...] + a1_ref[...] + y_ref[...]) + b_ref[...],
        0.0)
    bb = b3_ref[0, 0, :]
    ids = lax.broadcasted_iota(jnp.int32, (g, bn), 0)
    m = (ids == bb[None, :]).astype(jnp.float32)

    @pl.when(i == 0)
    def _():
      ps_ref[...] = jnp.zeros_like(ps_ref)
    ps_ref[...] += _DOT(m, hh)

  return pl.pallas_call(
      body,
      grid=(nb,),
      in_specs=[
          pl.BlockSpec((bn, h), lambda i: (i, 0)),
          pl.BlockSpec((bn, h), lambda i: (i, 0)),
          pl.BlockSpec((bn, h), lambda i: (i, 0)),
          pl.BlockSpec((bn, 1), lambda i: (i, 0)),
          pl.BlockSpec((1, h), lambda i: (0, 0)),
          pl.BlockSpec((1, 1, bn), lambda i: (i, 0, 0)),
      ],
      out_specs=pl.BlockSpec((g, h), lambda i: (0, 0)),
      out_shape=jax.ShapeDtypeStruct((g, h), jnp.float32),
  )(a0, a1, y, dis, bias, bat3)


def _tc_mlp(ps, cnt, l1w, l1b, l2w, l2b):
  g = ps.shape[1]

  def body(ps_ref, cnt_ref, w1_ref, b1_ref, w2_ref, b2_ref, o_ref):
    p = ps_ref[...]
    inv = 1.0 / jnp.maximum(cnt_ref[...], 1.0)
    hcat = jnp.concatenate([p[l] for l in range(p.shape[0])], axis=1) * inv
    t = jnp.maximum(_DOT(hcat, w1_ref[...]) + b1_ref[...], 0.0)
    o_ref[...] = _DOT(t, w2_ref[...]) + b2_ref[...]

  return pl.pallas_call(
      body, out_shape=jax.ShapeDtypeStruct((g, 1), jnp.float32),
  )(ps, cnt, l1w, l1b, l2w, l2b)


def kernel(x, edge_index, batch, W1, b1, W2, b2, W3, b3, W4, b4,
           L1W, L1b, L2W, L2b):
  n, d = x.shape
  h = W1.shape[1]
  bn = 1000
  src = edge_index[0]
  dst = edge_index[1]
  bat3 = batch.reshape(n // bn, 1, bn)
  zeros_n = jnp.zeros((n,), jnp.float32)
  zeros_nh = jnp.zeros((n, h), jnp.float32)
  chunk = 125
  srcm = src.reshape(-1, 1, chunk)
  dstm = dst.reshape(-1, 1, chunk)

  degp = _sc_degree(dst, zeros_n)
  dis = _tc_dis(degp)

  y, cnt = _tc_first(x, dis, W1, bat3, bn)

  biases = [b1.reshape(1, h), b2.reshape(1, h), b3.reshape(1, h),
            b4.reshape(1, h), b4.reshape(1, h)]
  wnexts = [W2, W3, W4, W4]

  psums = []
  for l in range(4):
    a0, a1 = _sc_agg(y, srcm, dstm, zeros_nh)
    y, ps = _tc_mid(a0, a1, y, dis, biases[l], wnexts[l], bat3, bn)
    psums.append(ps)

  a0, a1 = _sc_agg(y, srcm, dstm, zeros_nh)
  psums.append(_tc_last(a0, a1, y, dis, biases[4], bat3, bn))

  out = _tc_mlp(jnp.stack(psums), cnt, L1W, L1b.reshape(1, -1),
                L2W, L2b.reshape(1, 1))
  return out.reshape(-1)


# speedup vs baseline: 20.1429x; 1.0205x over previous
import functools

import jax
import jax.numpy as jnp
from jax import lax
from jax.experimental import pallas as pl
from jax.experimental.pallas import tpu as pltpu
from jax.experimental.pallas import tpu_sc as plsc

NC = 2
NS = 16

_MESH = plsc.VectorSubcoreMesh(
    core_axis_name="c", subcore_axis_name="s", num_cores=NC, num_subcores=NS)

_DOT = functools.partial(jnp.dot, preferred_element_type=jnp.float32)
_DOTX = functools.partial(
    jnp.dot, preferred_element_type=jnp.float32,
    precision=jax.lax.Precision.HIGHEST)


def _sc_degree(dst, zeros_n):
  e = dst.shape[0]
  n = zeros_n.shape[0]
  per_w = e // (NC * NS)
  chunk = 80
  nch = per_w // chunk
  assert per_w % chunk == 0 and per_w % 8 == 0

  @functools.partial(
      pl.kernel,
      out_type=[jax.ShapeDtypeStruct((n,), jnp.float32),
                jax.ShapeDtypeStruct((n,), jnp.float32)],
      mesh=_MESH,
      scratch_types=[
          pltpu.VMEM((chunk,), jnp.int32),
          pltpu.VMEM((chunk,), jnp.float32),
          pltpu.VMEM_SHARED((n,), jnp.float32),
      ],
  )
  def deg_kernel(dst_hbm, z_hbm, out0_hbm, out1_hbm, idx_v, ones_v, acc_sh):
    c = lax.axis_index("c")
    s = lax.axis_index("s")
    for j in range(chunk // 16):
      ones_v[pl.ds(j * 16, 16)] = jnp.full((16,), 1.0, jnp.float32)

    @pl.when(s == 0)
    def _():
      pltpu.sync_copy(z_hbm, acc_sh)
    plsc.subcore_barrier()

    wbase = (c * NS + s) * per_w

    def body(i, carry):
      base = pl.multiple_of(wbase + i * chunk, 8)
      pltpu.sync_copy(dst_hbm.at[pl.ds(base, chunk)], idx_v)
      pltpu.sync_copy(ones_v, acc_sh.at[idx_v], add=True)
      return carry

    lax.fori_loop(0, nch, body, 0)
    plsc.subcore_barrier()

    @pl.when((s == 0) & (c == 0))
    def _():
      pltpu.sync_copy(acc_sh, out0_hbm)

    @pl.when((s == 0) & (c == 1))
    def _():
      pltpu.sync_copy(acc_sh, out1_hbm)

  d0, d1 = deg_kernel(dst, zeros_n)
  return jnp.stack([d0, d1])


def _sc_agg(y, src, dst, zeros_nh):
  n, h = y.shape
  nrow, _, chunk = src.shape
  per_w = nrow // (NC * NS)
  assert per_w % 2 == 0
  nh = per_w // 2
  rows_pt = n // NS

  @functools.partial(
      pl.kernel,
      out_type=[jax.ShapeDtypeStruct((n, h), jnp.float32),
                jax.ShapeDtypeStruct((n, h), jnp.float32)],
      mesh=_MESH,
      scratch_types=[
          pltpu.VMEM((4, 1, chunk), jnp.int32),
          pltpu.VMEM((4, 1, chunk), jnp.int32),
          pltpu.VMEM((chunk, h), jnp.float32),
          pltpu.VMEM((chunk, h), jnp.float32),
          pltpu.VMEM_SHARED((n, h), jnp.float32),
          pltpu.SemaphoreType.DMA,
          pltpu.SemaphoreType.DMA,
          pltpu.SemaphoreType.DMA,
          pltpu.SemaphoreType.DMA,
          pltpu.SemaphoreType.DMA,
      ],
  )
  def agg_kernel(y_hbm, src_hbm, dst_hbm, z_hbm, out0_hbm, out1_hbm,
                 si_v, di_v, rows0_v, rows1_v, acc_sh, sem0, sem1, semi,
                 sems0, sems1):
    c = lax.axis_index("c")
    s = lax.axis_index("s")
    rw = (rows_pt // 8 + 1) * 8
    r0 = pl.multiple_of(s * rows_pt // 8 * 8, 8)
    pltpu.sync_copy(z_hbm.at[pl.ds(r0, rw)], acc_sh.at[pl.ds(r0, rw)])

    wr = (c * NS + s) * per_w
    pltpu.sync_copy(src_hbm.at[pl.ds(wr, 2)], si_v.at[pl.ds(0, 2)])
    pltpu.sync_copy(dst_hbm.at[pl.ds(wr, 2)], di_v.at[pl.ds(0, 2)])
    plsc.subcore_barrier()

    pltpu.async_copy(y_hbm.at[si_v.at[0, 0]], rows0_v, sem0)

    def body(k, carry):
      j = 2 * k
      s0 = j % 4
      s1 = (j + 1) % 4
      sp = (j + 2) % 4

      @pl.when(k > 0)
      def _():
        pltpu.make_async_copy(rows1_v, acc_sh.at[di_v.at[s1, 0]],
                              sems1).wait()

      pltpu.make_async_copy(y_hbm.at[si_v.at[s0, 0]], rows0_v, sem0).wait()
      pltpu.async_copy(y_hbm.at[si_v.at[s1, 0]], rows1_v, sem1)

      @pl.when(k < nh - 1)
      def _():
        pltpu.async_copy(src_hbm.at[pl.ds(wr + j + 2, 2)],
                         si_v.at[pl.ds(sp, 2)], semi)
        pltpu.async_copy(dst_hbm.at[pl.ds(wr + j + 2, 2)],
                         di_v.at[pl.ds(sp, 2)], semi)

      pltpu.async_copy(rows0_v, acc_sh.at[di_v.at[s0, 0]], sems0, add=True)
      pltpu.make_async_copy(y_hbm.at[si_v.at[s1, 0]], rows1_v, sem1).wait()

      @pl.when(k < nh - 1)
      def _():
        pltpu.make_async_copy(src_hbm.at[pl.ds(wr + j + 2, 2)],
                              si_v.at[pl.ds(sp, 2)], semi).wait()
        pltpu.make_async_copy(dst_hbm.at[pl.ds(wr + j + 2, 2)],
                              di_v.at[pl.ds(sp, 2)], semi).wait()

      pltpu.make_async_copy(rows0_v, acc_sh.at[di_v.at[s0, 0]], sems0).wait()

      @pl.when(k < nh - 1)
      def _():
        pltpu.async_copy(y_hbm.at[si_v.at[sp, 0]], rows0_v, sem0)

      pltpu.async_copy(rows1_v, acc_sh.at[di_v.at[s1, 0]], sems1, add=True)
      return carry

    lax.fori_loop(0, nh, body, 0)
    pltpu.make_async_copy(rows1_v, acc_sh.at[di_v.at[1, 0]], sems1).wait()
    plsc.subcore_barrier()

    @pl.when(c == 0)
    def _():
      pltpu.sync_copy(acc_sh.at[pl.ds(r0, rw)], out0_hbm.at[pl.ds(r0, rw)])

    @pl.when(c == 1)
    def _():
      pltpu.sync_copy(acc_sh.at[pl.ds(r0, rw)], out1_hbm.at[pl.ds(r0, rw)])

  return agg_kernel(y, src, dst, zeros_nh)


def _tc_dis(degp):
  n = degp.shape[1]

  def body(d_ref, o_ref):
    d = d_ref[...]
    o_ref[...] = lax.rsqrt(1.0 + d[0, :] + d[1, :])[:, None]

  return pl.pallas_call(
      body, out_shape=jax.ShapeDtypeStruct((n, 1), jnp.float32))(degp)


def _tc_first(x, dis, w1, bat3, bn):
  n, d = x.shape
  h = w1.shape[1]
  g = 64
  nb = n // bn

  def body(x_ref, dis_ref, w_ref, b3_ref, y_ref, cnt_ref):
    i = pl.program_id(0)
    y_ref[...] = dis_ref[...] * _DOT(x_ref[...], w_ref[...])
    bb = b3_ref[0, 0, :]
    ids = lax.broadcasted_iota(jnp.int32, (g, bn), 0)
    m = (ids == bb[None, :]).astype(jnp.float32)

    @pl.when(i == 0)
    def _():
      cnt_ref[...] = jnp.zeros_like(cnt_ref)
    cnt_ref[...] += jnp.sum(m, axis=1, keepdims=True)

  return pl.pallas_call(
      body,
      grid=(nb,),
      in_specs=[
          pl.BlockSpec((bn, d), lambda i: (i, 0)),
          pl.BlockSpec((bn, 1), lambda i: (i, 0)),
          pl.BlockSpec((d, h), lambda i: (0, 0)),
          pl.BlockSpec((1, 1, bn), lambda i: (i, 0, 0)),
      ],
      out_specs=[
          pl.BlockSpec((bn, h), lambda i: (i, 0)),
          pl.BlockSpec((g, 1), lambda i: (0, 0)),
      ],
      out_shape=[
          jax.ShapeDtypeStruct((n, h), jnp.float32),
          jax.ShapeDtypeStruct((g, 1), jnp.float32),
      ],
  )(x, dis, w1, bat3)


def _tc_mid(a0, a1, y, dis, bias, w_next, bat3, bn):
  n, h = y.shape
  g = 64
  nb = n // bn

  def body(a0_ref, a1_ref, y_ref, dis_ref, b_ref, w_ref, b3_ref,
           y2_ref, ps_ref):
    i = pl.program_id(0)
    dd = dis_ref[...]
    hh = jnp.maximum(
        dd * (a0_ref[...] + a1_ref[...] + y_ref[...]) + b_ref[...], 0.0)
    bb = b3_ref[0, 0, :]
    ids = lax.broadcasted_iota(jnp.int32, (g, bn), 0)
    m = (ids == bb[None, :]).astype(jnp.float32)

    @pl.when(i == 0)
    def _():
      ps_ref[...] = jnp.zeros_like(ps_ref)
    ps_ref[...] += _DOTX(m, hh)

    y2_ref[...] = dd * _DOT(hh, w_ref[...])

  return pl.pallas_call(
      body,
      grid=(nb,),
      in_specs=[
          pl.BlockSpec((bn, h), lambda i: (i, 0)),
          pl.BlockSpec((bn, h), lambda i: (i, 0)),
          pl.BlockSpec((bn, h), lambda i: (i, 0)),
          pl.BlockSpec((bn, 1), lambda i: (i, 0)),
          pl.BlockSpec((1, h), lambda i: (0, 0)),
          pl.BlockSpec((h, h), lambda i: (0, 0)),
          pl.BlockSpec((1, 1, bn), lambda i: (i, 0, 0)),
      ],
      out_specs=[
          pl.BlockSpec((bn, h), lambda i: (i, 0)),
          pl.BlockSpec((g, h), lambda i: (0, 0)),
      ],
      out_shape=[
          jax.ShapeDtypeStruct((n, h), jnp.float32),
          jax.ShapeDtypeStruct((g, h), jnp.float32),
      ],
  )(a0, a1, y, dis, bias, w_next, bat3)


def _tc_last(a0, a1, y, dis, bias, bat3, bn):
  n, h = y.shape
  g = 64
  nb = n // bn

  def body(a0_ref, a1_ref, y_ref, dis_ref, b_ref, b3_ref, ps_ref):
    i = pl.program_id(0)
    hh = jnp.maximum(
        dis_ref[...] * (a0_ref[...] + a1_ref[...] + y_ref[...]) + b_ref[...],
        0.0)
    bb = b3_ref[0, 0, :]
    ids = lax.broadcasted_iota(jnp.int32, (g, bn), 0)
    m = (ids == bb[None, :]).astype(jnp.float32)

    @pl.when(i == 0)
    def _():
      ps_ref[...] = jnp.zeros_like(ps_ref)
    ps_ref[...] += _DOTX(m, hh)

  return pl.pallas_call(
      body,
      grid=(nb,),
      in_specs=[
          pl.BlockSpec((bn, h), lambda i: (i, 0)),
          pl.BlockSpec((bn, h), lambda i: (i, 0)),
          pl.BlockSpec((bn, h), lambda i: (i, 0)),
          pl.BlockSpec((bn, 1), lambda i: (i, 0)),
          pl.BlockSpec((1, h), lambda i: (0, 0)),
          pl.BlockSpec((1, 1, bn), lambda i: (i, 0, 0)),
      ],
      out_specs=pl.BlockSpec((g, h), lambda i: (0, 0)),
      out_shape=jax.ShapeDtypeStruct((g, h), jnp.float32),
  )(a0, a1, y, dis, bias, bat3)


def _tc_mlp(ps, cnt, l1w, l1b, l2w, l2b):
  g = ps.shape[1]

  def body(ps_ref, cnt_ref, w1_ref, b1_ref, w2_ref, b2_ref, o_ref):
    p = ps_ref[...]
    inv = 1.0 / jnp.maximum(cnt_ref[...], 1.0)
    hcat = jnp.concatenate([p[l] for l in range(p.shape[0])], axis=1) * inv
    t = jnp.maximum(_DOT(hcat, w1_ref[...]) + b1_ref[...], 0.0)
    o_ref[...] = _DOT(t, w2_ref[...]) + b2_ref[...]

  return pl.pallas_call(
      body, out_shape=jax.ShapeDtypeStruct((g, 1), jnp.float32),
  )(ps, cnt, l1w, l1b, l2w, l2b)


def kernel(x, edge_index, batch, W1, b1, W2, b2, W3, b3, W4, b4,
           L1W, L1b, L2W, L2b):
  n, d = x.shape
  h = W1.shape[1]
  bn = 1000
  src = edge_index[0]
  dst = edge_index[1]
  bat3 = batch.reshape(n // bn, 1, bn)
  zeros_n = jnp.zeros((n,), jnp.float32)
  zeros_nh = jnp.zeros((n, h), jnp.float32)
  chunk = 125
  srcm = src.reshape(-1, 1, chunk)
  dstm = dst.reshape(-1, 1, chunk)

  degp = _sc_degree(dst, zeros_n)
  dis = _tc_dis(degp)

  y, cnt = _tc_first(x, dis, W1, bat3, bn)

  biases = [b1.reshape(1, h), b2.reshape(1, h), b3.reshape(1, h),
            b4.reshape(1, h), b4.reshape(1, h)]
  wnexts = [W2, W3, W4, W4]

  psums = []
  for l in range(4):
    a0, a1 = _sc_agg(y, srcm, dstm, zeros_nh)
    y, ps = _tc_mid(a0, a1, y, dis, biases[l], wnexts[l], bat3, bn)
    psums.append(ps)

  a0, a1 = _sc_agg(y, srcm, dstm, zeros_nh)
  psums.append(_tc_last(a0, a1, y, dis, biases[4], bat3, bn))

  out = _tc_mlp(jnp.stack(psums), cnt, L1W, L1b.reshape(1, -1),
                L2W, L2b.reshape(1, 1))
  return out.reshape(-1)


# speedup vs baseline: 20.4994x; 1.0177x over previous
import functools

import jax
import jax.numpy as jnp
from jax import lax
from jax.experimental import pallas as pl
from jax.experimental.pallas import tpu as pltpu
from jax.experimental.pallas import tpu_sc as plsc

NC = 2
NS = 16

_MESH = plsc.VectorSubcoreMesh(
    core_axis_name="c", subcore_axis_name="s", num_cores=NC, num_subcores=NS)

_DOT = functools.partial(jnp.dot, preferred_element_type=jnp.float32)
_DOTX = functools.partial(
    jnp.dot, preferred_element_type=jnp.float32,
    precision=jax.lax.Precision.HIGHEST)


def _sc_degree(dst, zeros_n):
  e = dst.shape[0]
  n = zeros_n.shape[0]
  per_w = e // (NC * NS)
  chunk = 80
  nch = per_w // chunk
  assert per_w % chunk == 0 and per_w % 8 == 0

  assert nch % 2 == 1

  @functools.partial(
      pl.kernel,
      out_type=[jax.ShapeDtypeStruct((n,), jnp.float32),
                jax.ShapeDtypeStruct((n,), jnp.float32)],
      mesh=_MESH,
      scratch_types=[
          pltpu.VMEM((chunk,), jnp.int32),
          pltpu.VMEM((chunk,), jnp.int32),
          pltpu.VMEM((chunk,), jnp.float32),
          pltpu.VMEM_SHARED((n,), jnp.float32),
          pltpu.SemaphoreType.DMA,
          pltpu.SemaphoreType.DMA,
      ],
  )
  def deg_kernel(dst_hbm, z_hbm, out0_hbm, out1_hbm, ia_v, ib_v, ones_v,
                 acc_sh, sema, semb):
    c = lax.axis_index("c")
    s = lax.axis_index("s")
    for j in range(chunk // 16):
      ones_v[pl.ds(j * 16, 16)] = jnp.full((16,), 1.0, jnp.float32)

    @pl.when(s == 0)
    def _():
      pltpu.sync_copy(z_hbm, acc_sh)
    plsc.subcore_barrier()

    wbase = (c * NS + s) * per_w
    pltpu.sync_copy(dst_hbm.at[pl.ds(pl.multiple_of(wbase, 8), chunk)], ia_v)
    pltpu.sync_copy(ones_v, acc_sh.at[ia_v], add=True)
    pltpu.async_copy(dst_hbm.at[pl.ds(pl.multiple_of(wbase + chunk, 8), chunk)],
                     ia_v, sema)

    def body(k, carry):
      j = 1 + 2 * k
      ba = pl.multiple_of(wbase + j * chunk, 8)
      bb = pl.multiple_of(wbase + (j + 1) * chunk, 8)
      bn2 = pl.multiple_of(wbase + (j + 2) * chunk, 8)
      pltpu.make_async_copy(dst_hbm.at[pl.ds(ba, chunk)], ia_v, sema).wait()
      pltpu.async_copy(dst_hbm.at[pl.ds(bb, chunk)], ib_v, semb)
      pltpu.sync_copy(ones_v, acc_sh.at[ia_v], add=True)
      pltpu.make_async_copy(dst_hbm.at[pl.ds(bb, chunk)], ib_v, semb).wait()

      @pl.when(k < (nch - 1) // 2 - 1)
      def _():
        pltpu.async_copy(dst_hbm.at[pl.ds(bn2, chunk)], ia_v, sema)
      pltpu.sync_copy(ones_v, acc_sh.at[ib_v], add=True)
      return carry

    lax.fori_loop(0, (nch - 1) // 2, body, 0)
    plsc.subcore_barrier()

    @pl.when((s == 0) & (c == 0))
    def _():
      pltpu.sync_copy(acc_sh, out0_hbm)

    @pl.when((s == 0) & (c == 1))
    def _():
      pltpu.sync_copy(acc_sh, out1_hbm)

  return deg_kernel(dst, zeros_n)


def _sc_agg(y, src, dst, zeros_nh):
  n, h = y.shape
  nrow, _, chunk = src.shape
  per_w = nrow // (NC * NS)
  assert per_w % 2 == 0
  nh = per_w // 2
  rows_pt = n // NS

  @functools.partial(
      pl.kernel,
      out_type=[jax.ShapeDtypeStruct((n, h), jnp.float32),
                jax.ShapeDtypeStruct((n, h), jnp.float32)],
      mesh=_MESH,
      scratch_types=[
          pltpu.VMEM((4, 1, chunk), jnp.int32),
          pltpu.VMEM((4, 1, chunk), jnp.int32),
          pltpu.VMEM((chunk, h), jnp.float32),
          pltpu.VMEM((chunk, h), jnp.float32),
          pltpu.VMEM_SHARED((n, h), jnp.float32),
          pltpu.SemaphoreType.DMA,
          pltpu.SemaphoreType.DMA,
          pltpu.SemaphoreType.DMA,
          pltpu.SemaphoreType.DMA,
          pltpu.SemaphoreType.DMA,
      ],
  )
  def agg_kernel(y_hbm, src_hbm, dst_hbm, z_hbm, out0_hbm, out1_hbm,
                 si_v, di_v, rows0_v, rows1_v, acc_sh, sem0, sem1, semi,
                 sems0, sems1):
    c = lax.axis_index("c")
    s = lax.axis_index("s")
    rw = (rows_pt // 8 + 1) * 8
    r0 = pl.multiple_of(s * rows_pt // 8 * 8, 8)
    pltpu.sync_copy(z_hbm.at[pl.ds(r0, rw)], acc_sh.at[pl.ds(r0, rw)])

    wr = (c * NS + s) * per_w
    pltpu.sync_copy(src_hbm.at[pl.ds(wr, 2)], si_v.at[pl.ds(0, 2)])
    pltpu.sync_copy(dst_hbm.at[pl.ds(wr, 2)], di_v.at[pl.ds(0, 2)])
    plsc.subcore_barrier()

    pltpu.async_copy(y_hbm.at[si_v.at[0, 0]], rows0_v, sem0)

    def body(k, carry):
      j = 2 * k
      s0 = j % 4
      s1 = (j + 1) % 4
      sp = (j + 2) % 4

      @pl.when(k > 0)
      def _():
        pltpu.make_async_copy(rows1_v, acc_sh.at[di_v.at[s1, 0]],
                              sems1).wait()

      pltpu.make_async_copy(y_hbm.at[si_v.at[s0, 0]], rows0_v, sem0).wait()
      pltpu.async_copy(y_hbm.at[si_v.at[s1, 0]], rows1_v, sem1)

      @pl.when(k < nh - 1)
      def _():
        pltpu.async_copy(src_hbm.at[pl.ds(wr + j + 2, 2)],
                         si_v.at[pl.ds(sp, 2)], semi)
        pltpu.async_copy(dst_hbm.at[pl.ds(wr + j + 2, 2)],
                         di_v.at[pl.ds(sp, 2)], semi)

      pltpu.async_copy(rows0_v, acc_sh.at[di_v.at[s0, 0]], sems0, add=True)
      pltpu.make_async_copy(y_hbm.at[si_v.at[s1, 0]], rows1_v, sem1).wait()

      @pl.when(k < nh - 1)
      def _():
        pltpu.make_async_copy(src_hbm.at[pl.ds(wr + j + 2, 2)],
                              si_v.at[pl.ds(sp, 2)], semi).wait()
        pltpu.make_async_copy(dst_hbm.at[pl.ds(wr + j + 2, 2)],
                              di_v.at[pl.ds(sp, 2)], semi).wait()

      pltpu.make_async_copy(rows0_v, acc_sh.at[di_v.at[s0, 0]], sems0).wait()

      @pl.when(k < nh - 1)
      def _():
        pltpu.async_copy(y_hbm.at[si_v.at[sp, 0]], rows0_v, sem0)

      pltpu.async_copy(rows1_v, acc_sh.at[di_v.at[s1, 0]], sems1, add=True)
      return carry

    lax.fori_loop(0, nh, body, 0)
    pltpu.make_async_copy(rows1_v, acc_sh.at[di_v.at[1, 0]], sems1).wait()
    plsc.subcore_barrier()

    @pl.when(c == 0)
    def _():
      pltpu.sync_copy(acc_sh.at[pl.ds(r0, rw)], out0_hbm.at[pl.ds(r0, rw)])

    @pl.when(c == 1)
    def _():
      pltpu.sync_copy(acc_sh.at[pl.ds(r0, rw)], out1_hbm.at[pl.ds(r0, rw)])

  return agg_kernel(y, src, dst, zeros_nh)


def _tc_first(x, d0, d1, w1, bat3, bn):
  n, d = x.shape
  h = w1.shape[1]
  g = 64
  nb = n // bn

  def body(x_ref, d0_ref, d1_ref, w_ref, b3_ref, y_ref, dis_ref, cnt_ref):
    i = pl.program_id(0)
    dd = lax.rsqrt(1.0 + d0_ref[0, 0, :] + d1_ref[0, 0, :])[:, None]
    dis_ref[...] = dd
    y_ref[...] = dd * _DOT(x_ref[...], w_ref[...])
    bb = b3_ref[0, 0, :]
    ids = lax.broadcasted_iota(jnp.int32, (g, bn), 0)
    m = (ids == bb[None, :]).astype(jnp.float32)

    @pl.when(i == 0)
    def _():
      cnt_ref[...] = jnp.zeros_like(cnt_ref)
    cnt_ref[...] += jnp.sum(m, axis=1, keepdims=True)

  return pl.pallas_call(
      body,
      grid=(nb,),
      in_specs=[
          pl.BlockSpec((bn, d), lambda i: (i, 0)),
          pl.BlockSpec((1, 1, bn), lambda i: (i, 0, 0)),
          pl.BlockSpec((1, 1, bn), lambda i: (i, 0, 0)),
          pl.BlockSpec((d, h), lambda i: (0, 0)),
          pl.BlockSpec((1, 1, bn), lambda i: (i, 0, 0)),
      ],
      out_specs=[
          pl.BlockSpec((bn, h), lambda i: (i, 0)),
          pl.BlockSpec((bn, 1), lambda i: (i, 0)),
          pl.BlockSpec((g, 1), lambda i: (0, 0)),
      ],
      out_shape=[
          jax.ShapeDtypeStruct((n, h), jnp.float32),
          jax.ShapeDtypeStruct((n, 1), jnp.float32),
          jax.ShapeDtypeStruct((g, 1), jnp.float32),
      ],
  )(x, d0.reshape(nb, 1, bn), d1.reshape(nb, 1, bn), w1, bat3)


def _tc_mid(a0, a1, y, dis, bias, w_next, bat3, bn):
  n, h = y.shape
  g = 64
  nb = n // bn

  def body(a0_ref, a1_ref, y_ref, dis_ref, b_ref, w_ref, b3_ref,
           y2_ref, ps_ref):
    i = pl.program_id(0)
    dd = dis_ref[...]
    hh = jnp.maximum(
        dd * (a0_ref[...] + a1_ref[...] + y_ref[...]) + b_ref[...], 0.0)
    bb = b3_ref[0, 0, :]
    ids = lax.broadcasted_iota(jnp.int32, (g, bn), 0)
    m = (ids == bb[None, :]).astype(jnp.float32)

    @pl.when(i == 0)
    def _():
      ps_ref[...] = jnp.zeros_like(ps_ref)
    ps_ref[...] += _DOTX(m, hh)

    y2_ref[...] = dd * _DOT(hh, w_ref[...])

  return pl.pallas_call(
      body,
      grid=(nb,),
      in_specs=[
          pl.BlockSpec((bn, h), lambda i: (i, 0)),
          pl.BlockSpec((bn, h), lambda i: (i, 0)),
          pl.BlockSpec((bn, h), lambda i: (i, 0)),
          pl.BlockSpec((bn, 1), lambda i: (i, 0)),
          pl.BlockSpec((1, h), lambda i: (0, 0)),
          pl.BlockSpec((h, h), lambda i: (0, 0)),
          pl.BlockSpec((1, 1, bn), lambda i: (i, 0, 0)),
      ],
      out_specs=[
          pl.BlockSpec((bn, h), lambda i: (i, 0)),
          pl.BlockSpec((g, h), lambda i: (0, 0)),
      ],
      out_shape=[
          jax.ShapeDtypeStruct((n, h), jnp.float32),
          jax.ShapeDtypeStruct((g, h), jnp.float32),
      ],
  )(a0, a1, y, dis, bias, w_next, bat3)


def _tc_last_mlp(a0, a1, y, dis, bias, bat3, psums, cnt, l1w, l1b, l2w, l2b,
                 bn):
  n, h = y.shape
  g = 64
  nb = n // bn

  def body(a0_ref, a1_ref, y_ref, dis_ref, b_ref, b3_ref,
           p1_ref, p2_ref, p3_ref, p4_ref, cnt_ref,
           w1_ref, b1_ref, w2_ref, b2_ref, o_ref, ps_ref):
    i = pl.program_id(0)
    hh = jnp.maximum(
        dis_ref[...] * (a0_ref[...] + a1_ref[...] + y_ref[...]) + b_ref[...],
        0.0)
    bb = b3_ref[0, 0, :]
    ids = lax.broadcasted_iota(jnp.int32, (g, bn), 0)
    m = (ids == bb[None, :]).astype(jnp.float32)

    @pl.when(i == 0)
    def _():
      ps_ref[...] = jnp.zeros_like(ps_ref)
    ps_ref[...] += _DOTX(m, hh)

    @pl.when(i == nb - 1)
    def _():
      inv = 1.0 / jnp.maximum(cnt_ref[...], 1.0)
      hcat = jnp.concatenate(
          [p1_ref[...], p2_ref[...], p3_ref[...], p4_ref[...], ps_ref[...]],
          axis=1) * inv
      t = jnp.maximum(_DOT(hcat, w1_ref[...]) + b1_ref[...], 0.0)
      o_ref[...] = _DOT(t, w2_ref[...]) + b2_ref[...]

  return pl.pallas_call(
      body,
      grid=(nb,),
      in_specs=[
          pl.BlockSpec((bn, h), lambda i: (i, 0)),
          pl.BlockSpec((bn, h), lambda i: (i, 0)),
          pl.BlockSpec((bn, h), lambda i: (i, 0)),
          pl.BlockSpec((bn, 1), lambda i: (i, 0)),
          pl.BlockSpec((1, h), lambda i: (0, 0)),
          pl.BlockSpec((1, 1, bn), lambda i: (i, 0, 0)),
          pl.BlockSpec((g, h), lambda i: (0, 0)),
          pl.BlockSpec((g, h), lambda i: (0, 0)),
          pl.BlockSpec((g, h), lambda i: (0, 0)),
          pl.BlockSpec((g, h), lambda i: (0, 0)),
          pl.BlockSpec((g, 1), lambda i: (0, 0)),
          pl.BlockSpec((5 * h, 5 * h), lambda i: (0, 0)),
          pl.BlockSpec((1, 5 * h), lambda i: (0, 0)),
          pl.BlockSpec((5 * h, 1), lambda i: (0, 0)),
          pl.BlockSpec((1, 1), lambda i: (0, 0)),
      ],
      out_specs=pl.BlockSpec((g, 1), lambda i: (0, 0)),
      out_shape=jax.ShapeDtypeStruct((g, 1), jnp.float32),
      scratch_shapes=[pltpu.VMEM((g, h), jnp.float32)],
  )(a0, a1, y, dis, bias, bat3, *psums, cnt, l1w, l1b, l2w, l2b)


def kernel(x, edge_index, batch, W1, b1, W2, b2, W3, b3, W4, b4,
           L1W, L1b, L2W, L2b):
  n, d = x.shape
  h = W1.shape[1]
  bn = 1000
  src = edge_index[0]
  dst = edge_index[1]
  bat3 = batch.reshape(n // bn, 1, bn)
  zeros_n = jnp.zeros((n,), jnp.float32)
  zeros_nh = jnp.zeros((n, h), jnp.float32)
  chunk = 125
  srcm = src.reshape(-1, 1, chunk)
  dstm = dst.reshape(-1, 1, chunk)

  d0, d1 = _sc_degree(dst, zeros_n)
  y, dis, cnt = _tc_first(x, d0, d1, W1, bat3, bn)

  biases = [b1.reshape(1, h), b2.reshape(1, h), b3.reshape(1, h),
            b4.reshape(1, h), b4.reshape(1, h)]
  wnexts = [W2, W3, W4, W4]

  psums = []
  for l in range(4):
    a0, a1 = _sc_agg(y, srcm, dstm, zeros_nh)
    y, ps = _tc_mid(a0, a1, y, dis, biases[l], wnexts[l], bat3, bn)
    psums.append(ps)

  a0, a1 = _sc_agg(y, srcm, dstm, zeros_nh)
  out = _tc_last_mlp(a0, a1, y, dis, biases[4], bat3, psums, cnt,
                     L1W, L1b.reshape(1, -1), L2W, L2b.reshape(1, 1), bn)
  return out.reshape(-1)


# speedup vs baseline: 20.8851x; 1.0188x over previous
import functools

import jax
import jax.numpy as jnp
from jax import lax
from jax.experimental import pallas as pl
from jax.experimental.pallas import tpu as pltpu
from jax.experimental.pallas import tpu_sc as plsc

NC = 2
NS = 16

_MESH = plsc.VectorSubcoreMesh(
    core_axis_name="c", subcore_axis_name="s", num_cores=NC, num_subcores=NS)

_DOT = functools.partial(jnp.dot, preferred_element_type=jnp.float32)
_DOTX = functools.partial(
    jnp.dot, preferred_element_type=jnp.float32,
    precision=jax.lax.Precision.HIGHEST)


def _sc_degree(dst, zeros_n):
  e = dst.shape[0]
  n = zeros_n.shape[0]
  per_w = e // (NC * NS)
  chunk = 80
  nch = per_w // chunk
  assert per_w % chunk == 0 and per_w % 8 == 0

  assert nch % 2 == 1

  @functools.partial(
      pl.kernel,
      out_type=[jax.ShapeDtypeStruct((n,), jnp.float32),
                jax.ShapeDtypeStruct((n,), jnp.float32)],
      mesh=_MESH,
      scratch_types=[
          pltpu.VMEM((chunk,), jnp.int32),
          pltpu.VMEM((chunk,), jnp.int32),
          pltpu.VMEM((chunk,), jnp.float32),
          pltpu.VMEM_SHARED((n,), jnp.float32),
          pltpu.SemaphoreType.DMA,
          pltpu.SemaphoreType.DMA,
      ],
  )
  def deg_kernel(dst_hbm, z_hbm, out0_hbm, out1_hbm, ia_v, ib_v, ones_v,
                 acc_sh, sema, semb):
    c = lax.axis_index("c")
    s = lax.axis_index("s")
    for j in range(chunk // 16):
      ones_v[pl.ds(j * 16, 16)] = jnp.full((16,), 1.0, jnp.float32)

    @pl.when(s == 0)
    def _():
      pltpu.sync_copy(z_hbm, acc_sh)
    plsc.subcore_barrier()

    wbase = (c * NS + s) * per_w
    pltpu.sync_copy(dst_hbm.at[pl.ds(pl.multiple_of(wbase, 8), chunk)], ia_v)
    pltpu.sync_copy(ones_v, acc_sh.at[ia_v], add=True)
    pltpu.async_copy(dst_hbm.at[pl.ds(pl.multiple_of(wbase + chunk, 8), chunk)],
                     ia_v, sema)

    def body(k, carry):
      j = 1 + 2 * k
      ba = pl.multiple_of(wbase + j * chunk, 8)
      bb = pl.multiple_of(wbase + (j + 1) * chunk, 8)
      bn2 = pl.multiple_of(wbase + (j + 2) * chunk, 8)
      pltpu.make_async_copy(dst_hbm.at[pl.ds(ba, chunk)], ia_v, sema).wait()
      pltpu.async_copy(dst_hbm.at[pl.ds(bb, chunk)], ib_v, semb)
      pltpu.sync_copy(ones_v, acc_sh.at[ia_v], add=True)
      pltpu.make_async_copy(dst_hbm.at[pl.ds(bb, chunk)], ib_v, semb).wait()

      @pl.when(k < (nch - 1) // 2 - 1)
      def _():
        pltpu.async_copy(dst_hbm.at[pl.ds(bn2, chunk)], ia_v, sema)
      pltpu.sync_copy(ones_v, acc_sh.at[ib_v], add=True)
      return carry

    lax.fori_loop(0, (nch - 1) // 2, body, 0)
    plsc.subcore_barrier()

    @pl.when((s == 0) & (c == 0))
    def _():
      pltpu.sync_copy(acc_sh, out0_hbm)

    @pl.when((s == 0) & (c == 1))
    def _():
      pltpu.sync_copy(acc_sh, out1_hbm)

  return deg_kernel(dst, zeros_n)


def _sc_agg(y, src, dst, zeros_nh):
  n, h = y.shape
  nrow, _, chunk = src.shape
  per_w = nrow // (NC * NS)
  assert per_w % 2 == 0
  nh = per_w // 2
  rows_pt = n // NS

  @functools.partial(
      pl.kernel,
      out_type=[jax.ShapeDtypeStruct((n, h), jnp.float32),
                jax.ShapeDtypeStruct((n, h), jnp.float32)],
      mesh=_MESH,
      scratch_types=[
          pltpu.VMEM((4, 1, chunk), jnp.int32),
          pltpu.VMEM((4, 1, chunk), jnp.int32),
          pltpu.VMEM((chunk, h), jnp.float32),
          pltpu.VMEM((chunk, h), jnp.float32),
          pltpu.VMEM_SHARED((n, h), jnp.float32),
          pltpu.SemaphoreType.DMA,
          pltpu.SemaphoreType.DMA,
          pltpu.SemaphoreType.DMA,
          pltpu.SemaphoreType.DMA,
          pltpu.SemaphoreType.DMA,
      ],
  )
  def agg_kernel(y_hbm, src_hbm, dst_hbm, z_hbm, out0_hbm, out1_hbm,
                 si_v, di_v, rows0_v, rows1_v, acc_sh, sem0, sem1, semi,
                 sems0, sems1):
    c = lax.axis_index("c")
    s = lax.axis_index("s")
    rw = (rows_pt // 8 + 1) * 8
    r0 = pl.multiple_of(s * rows_pt // 8 * 8, 8)
    pltpu.sync_copy(z_hbm.at[pl.ds(r0, rw)], acc_sh.at[pl.ds(r0, rw)])

    wr = (c * NS + s) * per_w
    pltpu.sync_copy(src_hbm.at[pl.ds(wr, 2)], si_v.at[pl.ds(0, 2)])
    pltpu.sync_copy(dst_hbm.at[pl.ds(wr, 2)], di_v.at[pl.ds(0, 2)])
    plsc.subcore_barrier()

    pltpu.async_copy(y_hbm.at[si_v.at[0, 0]], rows0_v, sem0)

    def body(k, carry):
      j = 2 * k
      s0 = j % 4
      s1 = (j + 1) % 4
      sp = (j + 2) % 4

      @pl.when(k > 0)
      def _():
        pltpu.make_async_copy(rows1_v, acc_sh.at[di_v.at[s1, 0]],
                              sems1).wait()

      pltpu.make_async_copy(y_hbm.at[si_v.at[s0, 0]], rows0_v, sem0).wait()
      pltpu.async_copy(y_hbm.at[si_v.at[s1, 0]], rows1_v, sem1)

      @pl.when(k < nh - 1)
      def _():
        pltpu.async_copy(src_hbm.at[pl.ds(wr + j + 2, 2)],
                         si_v.at[pl.ds(sp, 2)], semi)
        pltpu.async_copy(dst_hbm.at[pl.ds(wr + j + 2, 2)],
                         di_v.at[pl.ds(sp, 2)], semi)

      pltpu.async_copy(rows0_v, acc_sh.at[di_v.at[s0, 0]], sems0, add=True)
      pltpu.make_async_copy(y_hbm.at[si_v.at[s1, 0]], rows1_v, sem1).wait()

      @pl.when(k < nh - 1)
      def _():
        pltpu.make_async_copy(src_hbm.at[pl.ds(wr + j + 2, 2)],
                              si_v.at[pl.ds(sp, 2)], semi).wait()
        pltpu.make_async_copy(dst_hbm.at[pl.ds(wr + j + 2, 2)],
                              di_v.at[pl.ds(sp, 2)], semi).wait()

      pltpu.make_async_copy(rows0_v, acc_sh.at[di_v.at[s0, 0]], sems0).wait()

      @pl.when(k < nh - 1)
      def _():
        pltpu.async_copy(y_hbm.at[si_v.at[sp, 0]], rows0_v, sem0)

      pltpu.async_copy(rows1_v, acc_sh.at[di_v.at[s1, 0]], sems1, add=True)
      return carry

    lax.fori_loop(0, nh, body, 0)
    pltpu.make_async_copy(rows1_v, acc_sh.at[di_v.at[1, 0]], sems1).wait()
    plsc.subcore_barrier()

    @pl.when(c == 0)
    def _():
      pltpu.sync_copy(acc_sh.at[pl.ds(r0, rw)], out0_hbm.at[pl.ds(r0, rw)])

    @pl.when(c == 1)
    def _():
      pltpu.sync_copy(acc_sh.at[pl.ds(r0, rw)], out1_hbm.at[pl.ds(r0, rw)])

  return agg_kernel(y, src, dst, zeros_nh)


def _tc_first(x, d0, d1, w1, bat3, bn):
  n, d = x.shape
  h = w1.shape[1]
  g = 64
  nb = n // bn

  def body(x_ref, d0_ref, d1_ref, w_ref, b3_ref, y_ref, dis_ref, cnt_ref):
    i = pl.program_id(0)
    dd = lax.rsqrt(1.0 + d0_ref[0, 0, :] + d1_ref[0, 0, :])[:, None]
    dis_ref[...] = dd
    y_ref[...] = dd * _DOT(x_ref[...], w_ref[...])
    bb = b3_ref[0, 0, :]
    ids = lax.broadcasted_iota(jnp.int32, (g, bn), 0)
    m = (ids == bb[None, :]).astype(jnp.float32)

    @pl.when(i == 0)
    def _():
      cnt_ref[...] = jnp.zeros_like(cnt_ref)
    cnt_ref[...] += jnp.sum(m, axis=1, keepdims=True)

  return pl.pallas_call(
      body,
      grid=(nb,),
      in_specs=[
          pl.BlockSpec((bn, d), lambda i: (i, 0)),
          pl.BlockSpec((1, 1, bn), lambda i: (i, 0, 0)),
          pl.BlockSpec((1, 1, bn), lambda i: (i, 0, 0)),
          pl.BlockSpec((d, h), lambda i: (0, 0)),
          pl.BlockSpec((1, 1, bn), lambda i: (i, 0, 0)),
      ],
      out_specs=[
          pl.BlockSpec((bn, h), lambda i: (i, 0)),
          pl.BlockSpec((bn, 1), lambda i: (i, 0)),
          pl.BlockSpec((g, 1), lambda i: (0, 0)),
      ],
      out_shape=[
          jax.ShapeDtypeStruct((n, h), jnp.float32),
          jax.ShapeDtypeStruct((n, 1), jnp.float32),
          jax.ShapeDtypeStruct((g, 1), jnp.float32),
      ],
  )(x, d0.reshape(nb, 1, bn), d1.reshape(nb, 1, bn), w1, bat3)


def _tc_mid(a0, a1, y, dis, bias, w_next, bat3, bn):
  n, h = y.shape
  g = 64
  nb = n // bn

  def body(a0_ref, a1_ref, y_ref, dis_ref, b_ref, w_ref, b3_ref,
           y2_ref, ps_ref):
    i = pl.program_id(0)
    dd = dis_ref[...]
    hh = jnp.maximum(
        dd * (a0_ref[...] + a1_ref[...] + y_ref[...]) + b_ref[...], 0.0)
    bb = b3_ref[0, 0, :]
    ids = lax.broadcasted_iota(jnp.int32, (g, bn), 0)
    m = (ids == bb[None, :]).astype(jnp.float32)

    @pl.when(i == 0)
    def _():
      ps_ref[...] = jnp.zeros_like(ps_ref)
    ps_ref[...] += _DOTX(m, hh)

    y2_ref[...] = dd * _DOT(hh, w_ref[...])

  return pl.pallas_call(
      body,
      grid=(nb,),
      in_specs=[
          pl.BlockSpec((bn, h), lambda i: (i, 0)),
          pl.BlockSpec((bn, h), lambda i: (i, 0)),
          pl.BlockSpec((bn, h), lambda i: (i, 0)),
          pl.BlockSpec((bn, 1), lambda i: (i, 0)),
          pl.BlockSpec((1, h), lambda i: (0, 0)),
          pl.BlockSpec((h, h), lambda i: (0, 0)),
          pl.BlockSpec((1, 1, bn), lambda i: (i, 0, 0)),
      ],
      out_specs=[
          pl.BlockSpec((bn, h), lambda i: (i, 0)),
          pl.BlockSpec((g, h), lambda i: (0, 0)),
      ],
      out_shape=[
          jax.ShapeDtypeStruct((n, h), jnp.float32),
          jax.ShapeDtypeStruct((g, h), jnp.float32),
      ],
  )(a0, a1, y, dis, bias, w_next, bat3)


def _tc_last_mlp(a0, a1, y, dis, bias, bat3, psums, cnt, l1w, l1b, l2w, l2b,
                 bn):
  n, h = y.shape
  g = 64
  nb = n // bn

  def body(a0_ref, a1_ref, y_ref, dis_ref, b_ref, b3_ref,
           p1_ref, p2_ref, p3_ref, p4_ref, cnt_ref,
           w1_ref, b1_ref, w2_ref, b2_ref, o_ref, ps_ref):
    i = pl.program_id(0)
    hh = jnp.maximum(
        dis_ref[...] * (a0_ref[...] + a1_ref[...] + y_ref[...]) + b_ref[...],
        0.0)
    bb = b3_ref[0, 0, :]
    ids = lax.broadcasted_iota(jnp.int32, (g, bn), 0)
    m = (ids == bb[None, :]).astype(jnp.float32)

    @pl.when(i == 0)
    def _():
      ps_ref[...] = jnp.zeros_like(ps_ref)
    ps_ref[...] += _DOTX(m, hh)

    @pl.when(i == nb - 1)
    def _():
      inv = 1.0 / jnp.maximum(cnt_ref[...], 1.0)
      hcat = jnp.concatenate(
          [p1_ref[...], p2_ref[...], p3_ref[...], p4_ref[...], ps_ref[...]],
          axis=1) * inv
      t = jnp.maximum(_DOT(hcat, w1_ref[...]) + b1_ref[...], 0.0)
      o_ref[...] = _DOT(t, w2_ref[...]) + b2_ref[...]

  return pl.pallas_call(
      body,
      grid=(nb,),
      in_specs=[
          pl.BlockSpec((bn, h), lambda i: (i, 0)),
          pl.BlockSpec((bn, h), lambda i: (i, 0)),
          pl.BlockSpec((bn, h), lambda i: (i, 0)),
          pl.BlockSpec((bn, 1), lambda i: (i, 0)),
          pl.BlockSpec((1, h), lambda i: (0, 0)),
          pl.BlockSpec((1, 1, bn), lambda i: (i, 0, 0)),
          pl.BlockSpec((g, h), lambda i: (0, 0)),
          pl.BlockSpec((g, h), lambda i: (0, 0)),
          pl.BlockSpec((g, h), lambda i: (0, 0)),
          pl.BlockSpec((g, h), lambda i: (0, 0)),
          pl.BlockSpec((g, 1), lambda i: (0, 0)),
          pl.BlockSpec((5 * h, 5 * h), lambda i: (0, 0)),
          pl.BlockSpec((1, 5 * h), lambda i: (0, 0)),
          pl.BlockSpec((5 * h, 1), lambda i: (0, 0)),
          pl.BlockSpec((1, 1), lambda i: (0, 0)),
      ],
      out_specs=pl.BlockSpec((g, 1), lambda i: (0, 0)),
      out_shape=jax.ShapeDtypeStruct((g, 1), jnp.float32),
      scratch_shapes=[pltpu.VMEM((g, h), jnp.float32)],
  )(a0, a1, y, dis, bias, bat3, *psums, cnt, l1w, l1b, l2w, l2b)


def kernel(x, edge_index, batch, W1, b1, W2, b2, W3, b3, W4, b4,
           L1W, L1b, L2W, L2b):
  n, d = x.shape
  h = W1.shape[1]
  bn = 2000
  src = edge_index[0]
  dst = edge_index[1]
  bat3 = batch.reshape(n // bn, 1, bn)
  zeros_n = jnp.zeros((n,), jnp.float32)
  zeros_nh = jnp.zeros((n, h), jnp.float32)
  chunk = 125
  srcm = src.reshape(-1, 1, chunk)
  dstm = dst.reshape(-1, 1, chunk)

  d0, d1 = _sc_degree(dst, zeros_n)
  y, dis, cnt = _tc_first(x, d0, d1, W1, bat3, bn)

  biases = [b1.reshape(1, h), b2.reshape(1, h), b3.reshape(1, h),
            b4.reshape(1, h), b4.reshape(1, h)]
  wnexts = [W2, W3, W4, W4]

  psums = []
  for l in range(4):
    a0, a1 = _sc_agg(y, srcm, dstm, zeros_nh)
    y, ps = _tc_mid(a0, a1, y, dis, biases[l], wnexts[l], bat3, bn)
    psums.append(ps)

  a0, a1 = _sc_agg(y, srcm, dstm, zeros_nh)
  out = _tc_last_mlp(a0, a1, y, dis, biases[4], bat3, psums, cnt,
                     L1W, L1b.reshape(1, -1), L2W, L2b.reshape(1, 1), bn)
  return out.reshape(-1)
